# baseline (speedup 1.0000x reference)
import functools

import jax
import jax.numpy as jnp
from jax import lax
from jax.experimental import pallas as pl
from jax.experimental.pallas import tpu as pltpu
from jax.experimental.pallas import tpu_sc as plsc

N = 10000
NP = 10112
E = 320000
D = 128
EPS = 1e-5

NC = 2
NS = 16
NW = NC * NS
EPW = E // NW
CH = 176
NCH = -(-EPW // CH)
EPAD = NCH * CH - EPW
CB = 2 * CH
RPT = NP // NS

_mesh = plsc.VectorSubcoreMesh(core_axis_name="c", subcore_axis_name="s")


@functools.partial(
    pl.kernel,
    out_type=jax.ShapeDtypeStruct((NC, NP, D), jnp.float32),
    mesh=_mesh,
    scratch_types=[
        pltpu.VMEM((CB,), jnp.int32),
        pltpu.VMEM((CB,), jnp.int32),
        pltpu.VMEM((CH,), jnp.float32),
        pltpu.VMEM((CH,), jnp.float32),
        pltpu.VMEM((CH,), jnp.int32),
        pltpu.VMEM((CH,), jnp.int32),
        pltpu.VMEM((CH,), jnp.int32),
        pltpu.VMEM((CH,), jnp.int32),
        pltpu.VMEM((CH, D), jnp.float32),
        pltpu.VMEM((CH, D), jnp.float32),
        pltpu.VMEM_SHARED((NP, D), jnp.float32),
        pltpu.SemaphoreType.DMA,
        pltpu.SemaphoreType.DMA,
        pltpu.SemaphoreType.DMA,
        pltpu.SemaphoreType.DMA,
        pltpu.SemaphoreType.DMA,
        pltpu.SemaphoreType.DMA,
    ],
)
def _sc_aggregate(h_hbm, packed_hbm, wchunk_hbm, zero_hbm, out_hbm,
                  idx0, idx1, w0, w1, srcv0, srcv1, dsts0, dsts1,
                  rows0, rows1, aggr_s,
                  semi0, semi1, semg0, semg1, sems0, sems1):
    c = lax.axis_index("c")
    s = lax.axis_index("s")
    wid = c * NS + s

    idxb = (idx0, idx1)
    wb = (w0, w1)
    srcv = (srcv0, srcv1)
    dstsb = (dsts0, dsts1)
    rowsb = (rows0, rows1)
    semi = (semi0, semi1)
    semg = (semg0, semg1)
    sems = (sems0, sems1)

    pltpu.sync_copy(zero_hbm.at[pl.ds(s * RPT, RPT)],
                    aggr_s.at[pl.ds(s * RPT, RPT)])

    def idx_load(k, b):
        base = (wid * NCH + k) * CB
        wbase = (wid * NCH + k) * CH
        pltpu.async_copy(packed_hbm.at[pl.ds(base, CB)], idxb[b], semi[b])
        pltpu.async_copy(wchunk_hbm.at[pl.ds(wbase, CH)], wb[b], semi[b])

    def idx_wait(b):
        pltpu.make_async_copy(packed_hbm.at[pl.ds(0, CB)], idxb[b],
                              semi[b]).wait()
        pltpu.make_async_copy(wchunk_hbm.at[pl.ds(0, CH)], wb[b],
                              semi[b]).wait()

    def src_copy(b):
        def tbody(g, carry):
            sl = pl.ds(g * 16, 16)
            srcv[b][sl] = idxb[b][sl]
            return carry
        lax.fori_loop(0, CH // 16, tbody, 0)

    def gather_start(b):
        pltpu.async_copy(h_hbm.at[srcv[b]], rowsb[b], semg[b])

    def gather_wait(b):
        pltpu.make_async_copy(h_hbm.at[srcv[b]], rowsb[b], semg[b]).wait()

    def scale_and_copy(b):
        def gbody(g, carry):
            e0 = g * 16
            w16 = wb[b][pl.ds(e0, 16)]
            dstsb[b][pl.ds(e0, 16)] = idxb[b][pl.ds(CH + e0, 16)]
            for i in range(16):
                wv = jnp.full((16,), w16[i], dtype=jnp.float32)
                for j in range(D // 16):
                    sl = pl.ds(j * 16, 16)
                    rowsb[b][e0 + i, sl] = rowsb[b][e0 + i, sl] * wv
            return carry
        lax.fori_loop(0, CH // 16, gbody, 0)

    def scatter_start(b):
        pltpu.async_copy(rowsb[b], aggr_s.at[dstsb[b]], sems[b], add=True)

    def scatter_wait(b):
        pltpu.make_async_copy(rowsb[b], aggr_s.at[dstsb[b]], sems[b]).wait()

    def chunk_step(k, b, nb, prefetch):
        gather_wait(b)
        scale_and_copy(b)
        scatter_start(b)

        @pl.when(k < NCH - 2)
        def _():
            idx_load(k + 2, b)

        if prefetch:
            @pl.when(k > 0)
            def _():
                scatter_wait(nb)
            idx_wait(nb)
            src_copy(nb)
            gather_start(nb)

    idx_load(0, 0)
    idx_load(1, 1)
    idx_wait(0)
    src_copy(0)
    plsc.subcore_barrier()
    gather_start(0)

    def group(g, carry):
        for b in range(2):
            k = g * 2 + b
            chunk_step(k, b, 1 - b, prefetch=True)
        return carry

    lax.fori_loop(0, (NCH - 1) // 2, group, 0)

    chunk_step(NCH - 1, (NCH - 1) % 2, NCH % 2, prefetch=False)
    scatter_wait(0)
    scatter_wait(1)
    plsc.subcore_barrier()
    pltpu.sync_copy(aggr_s.at[pl.ds(s * RPT, RPT)],
                    out_hbm.at[c, pl.ds(s * RPT, RPT)])


_BN = 2528


def _tc_body(p_ref, w_ref, b_ref, o_ref):
    x = p_ref[0] + p_ref[1]
    y = jnp.dot(x, w_ref[...], preferred_element_type=jnp.float32)
    o_ref[...] = jnp.maximum(y + b_ref[...], 0.0)


def _tc_update(part, wt, bias):
    return pl.pallas_call(
        _tc_body,
        out_shape=jax.ShapeDtypeStruct((NP, D), jnp.float32),
        grid=(NP // _BN,),
        in_specs=[
            pl.BlockSpec((NC, _BN, D), lambda i: (0, i, 0)),
            pl.BlockSpec((D, D), lambda i: (0, 0)),
            pl.BlockSpec((1, D), lambda i: (0, 0)),
        ],
        out_specs=pl.BlockSpec((_BN, D), lambda i: (i, 0)),
    )(part, wt, bias)


def _pack_indices(src, dst, w):
    pad = ((0, 0), (0, EPAD))
    srcp = jnp.pad(src.reshape(NW, EPW), pad)
    dstp = jnp.pad(dst.reshape(NW, EPW), pad, constant_values=NP - 1)
    wp = jnp.pad(w.reshape(NW, EPW), pad)
    blocks = jnp.stack([srcp.reshape(NW, NCH, CH),
                        dstp.reshape(NW, NCH, CH)], axis=2)
    return blocks.reshape(-1), wp.reshape(-1)


def kernel(node_features, edge_index, edge_weights,
           W0, b0, gamma0, beta0,
           W1, b1, gamma1, beta1,
           W2, b2, gamma2, beta2):
    src = edge_index[0]
    dst = edge_index[1]
    packed, wchunk = _pack_indices(src, dst, edge_weights)
    zero = jnp.zeros((NP, D), jnp.float32)
    scale = 1.0 / jnp.sqrt(jnp.float32(1.0) + EPS)
    h = jnp.concatenate(
        [node_features, jnp.zeros((NP - N, D), jnp.float32)], axis=0)
    for W, b, g, bt in ((W0, b0, gamma0, beta0),
                        (W1, b1, gamma1, beta1),
                        (W2, b2, gamma2, beta2)):
        geff = g * scale
        wt = (W * geff[:, None]).T
        bias = (b * geff + bt)[None, :]
        part = _sc_aggregate(h, packed, wchunk, zero)
        h = _tc_update(part, wt, bias)
    return h[:N]


# speedup vs baseline: 1.6320x; 1.6320x over previous
import functools

import jax
import jax.numpy as jnp
from jax import lax
from jax.experimental import pallas as pl
from jax.experimental.pallas import tpu as pltpu
from jax.experimental.pallas import tpu_sc as plsc

N = 10000
NP = 10112
E = 320000
D = 128
EPS = 1e-5

NC = 2
NS = 16
NW = NC * NS
EPW = E // NW
CH = 80
NCH = EPW // CH
CB = 2 * CH
NB = 4
RPT = NP // NS

_mesh = plsc.VectorSubcoreMesh(core_axis_name="c", subcore_axis_name="s")

_scratch = ([pltpu.VMEM((CB,), jnp.int32) for _ in range(NB)] +
            [pltpu.VMEM((CH,), jnp.float32) for _ in range(NB)] +
            [pltpu.VMEM((CH,), jnp.int32) for _ in range(NB)] +
            [pltpu.VMEM((CH,), jnp.int32) for _ in range(NB)] +
            [pltpu.VMEM((CH, D), jnp.float32) for _ in range(NB)] +
            [pltpu.VMEM_SHARED((NP, D), jnp.float32)] +
            [pltpu.SemaphoreType.DMA for _ in range(3 * NB)])


@functools.partial(
    pl.kernel,
    out_type=jax.ShapeDtypeStruct((NC, NP, D), jnp.float32),
    mesh=_mesh,
    scratch_types=_scratch,
)
def _sc_aggregate(h_hbm, packed_hbm, wchunk_hbm, zero_hbm, out_hbm, *refs):
    idxb = refs[0:NB]
    wb = refs[NB:2 * NB]
    srcv = refs[2 * NB:3 * NB]
    dstsb = refs[3 * NB:4 * NB]
    rowsb = refs[4 * NB:5 * NB]
    aggr_s = refs[5 * NB]
    semi = refs[5 * NB + 1:5 * NB + 1 + NB]
    semg = refs[5 * NB + 1 + NB:5 * NB + 1 + 2 * NB]
    sems = refs[5 * NB + 1 + 2 * NB:5 * NB + 1 + 3 * NB]

    c = lax.axis_index("c")
    s = lax.axis_index("s")
    wid = c * NS + s

    pltpu.sync_copy(zero_hbm.at[pl.ds(s * RPT, RPT)],
                    aggr_s.at[pl.ds(s * RPT, RPT)])

    def idx_load(k, b):
        base = (wid * NCH + k) * CB
        wbase = (wid * NCH + k) * CH
        pltpu.async_copy(packed_hbm.at[pl.ds(base, CB)], idxb[b], semi[b])
        pltpu.async_copy(wchunk_hbm.at[pl.ds(wbase, CH)], wb[b], semi[b])

    def idx_wait(b):
        pltpu.make_async_copy(packed_hbm.at[pl.ds(0, CB)], idxb[b],
                              semi[b]).wait()
        pltpu.make_async_copy(wchunk_hbm.at[pl.ds(0, CH)], wb[b],
                              semi[b]).wait()

    def src_copy(b):
        def tbody(g, carry):
            sl = pl.ds(g * 16, 16)
            srcv[b][sl] = idxb[b][sl]
            return carry
        lax.fori_loop(0, CH // 16, tbody, 0)

    def gather_start(b):
        pltpu.async_copy(h_hbm.at[srcv[b]], rowsb[b], semg[b])

    def gather_wait(b):
        pltpu.make_async_copy(h_hbm.at[srcv[b]], rowsb[b], semg[b]).wait()

    def scale_and_copy(b):
        def gbody(g, carry):
            e0 = g * 16
            w16 = wb[b][pl.ds(e0, 16)]
            dstsb[b][pl.ds(e0, 16)] = idxb[b][pl.ds(CH + e0, 16)]
            for i in range(16):
                wv = jnp.full((16,), w16[i], dtype=jnp.float32)
                for j in range(D // 16):
                    sl = pl.ds(j * 16, 16)
                    rowsb[b][e0 + i, sl] = rowsb[b][e0 + i, sl] * wv
            return carry
        lax.fori_loop(0, CH // 16, gbody, 0)

    def scatter_start(b):
        pltpu.async_copy(rowsb[b], aggr_s.at[dstsb[b]], sems[b], add=True)

    def scatter_wait(b):
        pltpu.make_async_copy(rowsb[b], aggr_s.at[dstsb[b]], sems[b]).wait()

    def chunk_step(k, b, prefetch):
        gather_wait(b)
        scale_and_copy(b)
        scatter_start(b)

        @pl.when(k < NCH - NB)
        def _():
            idx_load(k + NB, b)

        if prefetch:
            p = (b + NB - 1) % NB

            @pl.when(k > 0)
            def _():
                scatter_wait(p)

            @pl.when(k < NCH - (NB - 1))
            def _():
                idx_wait(p)
                src_copy(p)
                gather_start(p)

    for b in range(NB):
        idx_load(b, b)
    for b in range(NB - 1):
        idx_wait(b)
        src_copy(b)
    plsc.subcore_barrier()
    for b in range(NB - 1):
        gather_start(b)

    def group(g, carry):
        for b in range(NB):
            k = g * NB + b
            chunk_step(k, b, prefetch=True)
        return carry

    lax.fori_loop(0, (NCH - 1) // NB, group, 0)

    chunk_step(NCH - 1, (NCH - 1) % NB, prefetch=False)
    scatter_wait((NCH - 2) % NB)
    scatter_wait((NCH - 1) % NB)
    plsc.subcore_barrier()
    pltpu.sync_copy(aggr_s.at[pl.ds(s * RPT, RPT)],
                    out_hbm.at[c, pl.ds(s * RPT, RPT)])


_BN = 2528


def _tc_body(p_ref, w_ref, b_ref, o_ref):
    x = p_ref[0] + p_ref[1]
    y = jnp.dot(x, w_ref[...], preferred_element_type=jnp.float32)
    o_ref[...] = jnp.maximum(y + b_ref[...], 0.0)


def _tc_update(part, wt, bias):
    return pl.pallas_call(
        _tc_body,
        out_shape=jax.ShapeDtypeStruct((NP, D), jnp.float32),
        grid=(NP // _BN,),
        in_specs=[
            pl.BlockSpec((NC, _BN, D), lambda i: (0, i, 0)),
            pl.BlockSpec((D, D), lambda i: (0, 0)),
            pl.BlockSpec((1, D), lambda i: (0, 0)),
        ],
        out_specs=pl.BlockSpec((_BN, D), lambda i: (i, 0)),
    )(part, wt, bias)


def _pack_indices(src, dst):
    srcp = src.reshape(NW, NCH, CH)
    dstp = dst.reshape(NW, NCH, CH)
    return jnp.stack([srcp, dstp], axis=2).reshape(-1)


def kernel(node_features, edge_index, edge_weights,
           W0, b0, gamma0, beta0,
           W1, b1, gamma1, beta1,
           W2, b2, gamma2, beta2):
    src = edge_index[0]
    dst = edge_index[1]
    packed = _pack_indices(src, dst)
    zero = jnp.zeros((NP, D), jnp.float32)
    scale = 1.0 / jnp.sqrt(jnp.float32(1.0) + EPS)
    h = jnp.concatenate(
        [node_features, jnp.zeros((NP - N, D), jnp.float32)], axis=0)
    for W, b, g, bt in ((W0, b0, gamma0, beta0),
                        (W1, b1, gamma1, beta1),
                        (W2, b2, gamma2, beta2)):
        geff = g * scale
        wt = (W * geff[:, None]).T
        bias = (b * geff + bt)[None, :]
        part = _sc_aggregate(h, packed, edge_weights, zero)
        h = _tc_update(part, wt, bias)
    return h[:N]
